# baseline (speedup 1.0000x reference)
import functools

import jax
import jax.numpy as jnp
from jax import lax
from jax.experimental import pallas as pl
from jax.experimental.pallas import tpu as pltpu
from jax.experimental.pallas import tpu_sc as plsc

N_PAD = 10240
E_PAD = 327680
CH = 128
RB = 1280

_info = plsc.get_sparse_core_info()
NC, NS = _info.num_cores, _info.num_subcores
NW = NC * NS
ECH = E_PAD // CH
ECH_STAGE = ECH + 32
NCHUNK = ECH // NW
ROWS_PT = N_PAD // NS

NCH0 = 144
NCH1 = 16
PH_BUF = 24

_MESH = dict(mesh=plsc.VectorSubcoreMesh(core_axis_name="c", subcore_axis_name="s"))


@functools.partial(
    pl.kernel,
    out_type=jax.ShapeDtypeStruct((NC * N_PAD,), jnp.float32),
    scratch_types=[
        pltpu.VMEM((NCHUNK, CH), jnp.int32),
        pltpu.VMEM((CH,), jnp.float32),
        pltpu.VMEM((640,), jnp.float32),
        pltpu.VMEM_SHARED((N_PAD,), jnp.float32),
        pltpu.SemaphoreType.DMA,
    ],
    **_MESH,
)
def _deg_k(dst_hbm, out_hbm, dsts, onesb, zbuf, acc, sem):
    cid = lax.axis_index("c")
    sid = lax.axis_index("s")
    wid = sid * NC + cid
    for j in range(CH // 16):
        onesb[pl.ds(j * 16, 16)] = jnp.ones((16,), jnp.float32)
    for j in range(640 // 16):
        zbuf[pl.ds(j * 16, 16)] = jnp.zeros((16,), jnp.float32)
    base_r = sid * ROWS_PT
    pltpu.sync_copy(dst_hbm.at[pl.ds(wid * NCHUNK, NCHUNK)], dsts)
    pltpu.sync_copy(zbuf.at[pl.ds(0, ROWS_PT)], acc.at[pl.ds(base_r, ROWS_PT)])
    plsc.subcore_barrier()

    def body(g, carry):
        pltpu.async_copy(onesb, acc.at[dsts.at[g]], sem, add=True)
        return carry

    lax.fori_loop(0, NCHUNK, body, 0)

    def drain(g, carry):
        pltpu.make_async_copy(onesb, acc.at[dsts.at[g]], sem).wait()
        return carry

    lax.fori_loop(0, NCHUNK, drain, 0)
    plsc.subcore_barrier()
    obase = pl.multiple_of(cid * N_PAD + base_r, 8)
    pltpu.sync_copy(acc.at[pl.ds(base_r, ROWS_PT)],
                    out_hbm.at[pl.ds(obase, ROWS_PT)])


CUPH = 10
CUCH = 16


@functools.partial(
    pl.kernel,
    out_type=[jax.ShapeDtypeStruct((NC, N_PAD, 128), jnp.float32),
              jax.ShapeDtypeStruct((NC, N_PAD, 128), jnp.float32),
              jax.ShapeDtypeStruct((2 * N_PAD,), jnp.float32)],
    scratch_types=[
        pltpu.VMEM((PH_BUF, CH), jnp.int32),
        pltpu.VMEM((PH_BUF, CH), jnp.int32),
        pltpu.VMEM((CH, 128), jnp.float32),
        pltpu.VMEM((CH, 128), jnp.float32),
        pltpu.VMEM((CH,), jnp.float32),
        pltpu.VMEM((CH,), jnp.float32),
        pltpu.VMEM((CH,), jnp.float32),
        pltpu.VMEM_SHARED((N_PAD, 128), jnp.float32),
        pltpu.VMEM_SHARED((N_PAD,), jnp.float32),
        pltpu.VMEM_SHARED((N_PAD,), jnp.float32),
        pltpu.SemaphoreType.DMA,
        pltpu.SemaphoreType.DMA,
        pltpu.SemaphoreType.DMA,
        pltpu.SemaphoreType.DMA,
        pltpu.SemaphoreType.DMA,
        pltpu.SemaphoreType.DMA,
        pltpu.SemaphoreType.DMA,
        pltpu.SemaphoreType.DMA,
        pltpu.SemaphoreType.DMA,
        pltpu.SemaphoreType.DMA,
    ],
    **_MESH,
)
def _edge_k(Y_hbm, X_hbm, row_hbm, col_hbm, q_hbm, zero_hbm, zvec_hbm,
            S_out, T_out, cu_out, srcs, dsts, rows0, rows1, onesb, ust0, ust1,
            acc, acc_c, acc_u, gsem0, gsem1, ssem0, ssem1, zsem, csem,
            qsem0, qsem1, usem0, usem1):
    cid = lax.axis_index("c")
    sid = lax.axis_index("s")
    is0 = cid == 0
    base_r = sid * ROWS_PT

    def zero_acc():
        pltpu.async_copy(zero_hbm.at[pl.ds(base_r, ROWS_PT)],
                         acc.at[pl.ds(base_r, ROWS_PT)], zsem)
        pltpu.make_async_copy(zero_hbm.at[pl.ds(base_r, ROWS_PT)],
                              acc.at[pl.ds(base_r, ROWS_PT)], zsem).wait()

    zero_acc()

    @pl.when(jnp.logical_not(is0))
    def _():
        for j in range(CH // 16):
            onesb[pl.ds(j * 16, 16)] = jnp.ones((16,), jnp.float32)
        pltpu.sync_copy(zvec_hbm.at[pl.ds(base_r, ROWS_PT)],
                        acc_c.at[pl.ds(base_r, ROWS_PT)])
        pltpu.sync_copy(zvec_hbm.at[pl.ds(base_r, ROWS_PT)],
                        acc_u.at[pl.ds(base_r, ROWS_PT)])

    plsc.subcore_barrier()

    def run_phase(table_hbm, src_hbm, dst_hbm, pstart, ph):
        pltpu.sync_copy(src_hbm.at[pl.ds(pstart, PH_BUF)], srcs)
        pltpu.sync_copy(dst_hbm.at[pl.ds(pstart, PH_BUF)], dsts)
        pltpu.async_copy(table_hbm.at[srcs.at[0]], rows0, gsem0)

        def body(j, carry):
            g = 2 * j
            pltpu.make_async_copy(table_hbm.at[srcs.at[g]], rows0, gsem0).wait()
            pltpu.async_copy(rows0, acc.at[dsts.at[g]], ssem0, add=True)

            @pl.when(j > 0)
            def _():
                pltpu.make_async_copy(rows1, acc.at[dsts.at[g - 1]], ssem1).wait()

            pltpu.async_copy(table_hbm.at[srcs.at[g + 1]], rows1, gsem1)
            pltpu.make_async_copy(table_hbm.at[srcs.at[g + 1]], rows1, gsem1).wait()
            pltpu.async_copy(rows1, acc.at[dsts.at[g + 1]], ssem1, add=True)
            pltpu.make_async_copy(rows0, acc.at[dsts.at[g]], ssem0).wait()

            @pl.when(2 * j + 2 < ph)
            def _():
                pltpu.async_copy(table_hbm.at[srcs.at[g + 2]], rows0, gsem0)

            return carry

        lax.fori_loop(0, ph // 2, body, 0)
        pltpu.make_async_copy(rows1, acc.at[dsts.at[ph - 1]], ssem1).wait()

    def run_pass(table_hbm, src_hbm, dst_hbm, out_hbm):
        @pl.when(is0)
        def _():
            for p in range(NCH0 // PH_BUF):
                run_phase(table_hbm, src_hbm, dst_hbm,
                          pl.multiple_of(sid * NCH0 + p * PH_BUF, 8), PH_BUF)

        @pl.when(jnp.logical_not(is0))
        def _():
            run_phase(table_hbm, src_hbm, dst_hbm,
                      pl.multiple_of(NS * NCH0 + sid * NCH1, 8), NCH1)

        plsc.subcore_barrier()
        pltpu.sync_copy(acc.at[pl.ds(base_r, ROWS_PT)],
                        out_hbm.at[cid, pl.ds(base_r, ROWS_PT)])

    run_pass(Y_hbm, row_hbm, col_hbm, S_out)
    zero_acc()
    plsc.subcore_barrier()
    run_pass(X_hbm, col_hbm, row_hbm, T_out)

    @pl.when(jnp.logical_not(is0))
    def _():
        for cp in range(CUPH):
            cstart = sid * (CUPH * CUCH) + cp * CUCH
            pltpu.sync_copy(row_hbm.at[pl.ds(cstart, CUCH)], srcs.at[pl.ds(0, CUCH)])
            pltpu.sync_copy(col_hbm.at[pl.ds(cstart, CUCH)], dsts.at[pl.ds(0, CUCH)])
            pltpu.async_copy(q_hbm.at[dsts.at[0]], ust0, qsem0)

            def ubody(j, carry):
                g = 2 * j
                pltpu.async_copy(onesb, acc_c.at[srcs.at[g]], csem, add=True)
                pltpu.async_copy(onesb, acc_c.at[srcs.at[g + 1]], csem, add=True)
                pltpu.make_async_copy(q_hbm.at[dsts.at[g]], ust0, qsem0).wait()
                pltpu.async_copy(ust0, acc_u.at[srcs.at[g]], usem0, add=True)

                @pl.when(j > 0)
                def _():
                    pltpu.make_async_copy(ust1, acc_u.at[srcs.at[g - 1]], usem1).wait()

                pltpu.async_copy(q_hbm.at[dsts.at[g + 1]], ust1, qsem1)
                pltpu.make_async_copy(q_hbm.at[dsts.at[g + 1]], ust1, qsem1).wait()
                pltpu.async_copy(ust1, acc_u.at[srcs.at[g + 1]], usem1, add=True)
                pltpu.make_async_copy(ust0, acc_u.at[srcs.at[g]], usem0).wait()

                @pl.when(2 * j + 2 < CUCH)
                def _():
                    pltpu.async_copy(q_hbm.at[dsts.at[g + 2]], ust0, qsem0)

                return carry

            lax.fori_loop(0, CUCH // 2, ubody, 0)
            pltpu.make_async_copy(ust1, acc_u.at[srcs.at[CUCH - 1]], usem1).wait()

            def cdrain(g, carry):
                pltpu.make_async_copy(onesb, acc_c.at[srcs.at[g]], csem).wait()
                return carry

            lax.fori_loop(0, CUCH, cdrain, 0)
        plsc.subcore_barrier()
        obase = pl.multiple_of(base_r, 8)
        pltpu.sync_copy(acc_c.at[pl.ds(base_r, ROWS_PT)],
                        cu_out.at[pl.ds(obase, ROWS_PT)])
        obase_u = pl.multiple_of(N_PAD + base_r, 8)
        pltpu.sync_copy(acc_u.at[pl.ds(base_r, ROWS_PT)],
                        cu_out.at[pl.ds(obase_u, ROWS_PT)])


def _enc_body(x_ref, we_ref, be_ref, ws_ref, X_ref, skip_ref):
    X = jnp.maximum(
        jnp.dot(x_ref[...], we_ref[...], preferred_element_type=jnp.float32)
        + be_ref[...], 0.0)
    X_ref[...] = X
    skip_ref[...] = jnp.dot(X, ws_ref[...], preferred_element_type=jnp.float32)


def _pre_body(X_ref, deg_ref, wc_ref, Y_ref, q_ref):
    dinv = lax.rsqrt(deg_ref[...] + 1.0)
    X = X_ref[...]
    XW = jnp.dot(X, wc_ref[...], preferred_element_type=jnp.float32)
    Y_ref[...] = dinv * XW
    q_ref[...] = jnp.sum(X * X, axis=1, keepdims=True)


def _post_body(X_ref, Y_ref, skip_ref, deg_ref, cnt_ref, u_ref, S_ref, T_ref,
               bc_ref, Xn_ref):
    dinv = lax.rsqrt(deg_ref[...] + 1.0)
    X = X_ref[...]
    S = S_ref[0] + S_ref[1]
    Xagg = jnp.maximum(dinv * (S + Y_ref[...]) + bc_ref[...], 0.0)
    T = T_ref[0] + T_ref[1]
    cnt = cnt_ref[...]
    q = jnp.sum(X * X, axis=1, keepdims=True)
    sd = cnt * q + u_ref[...] - 2.0 * jnp.sum(X * T, axis=1, keepdims=True)
    g = jnp.tanh(sd / jnp.maximum(cnt, 1.0))
    Xn_ref[...] = (X + g * (Xagg + skip_ref[...])) / (1.0 + 2.0 * g)


def _dec_body(X_ref, wd_ref, bd_ref, out_ref):
    out_ref[...] = (
        jnp.dot(X_ref[...], wd_ref[...], preferred_element_type=jnp.float32)
        + bd_ref[...])


def _rows_spec(w):
    return pl.BlockSpec((RB, w), lambda i: (i, 0))


def _full_spec(shape):
    return pl.BlockSpec(shape, lambda i: tuple(0 for _ in shape))


def _part_spec(w):
    return pl.BlockSpec((NC, RB, w), lambda i: (0, i, 0))


_GRID = N_PAD // RB

_enc = pl.pallas_call(
    _enc_body,
    grid=(_GRID,),
    in_specs=[_rows_spec(128), _full_spec((128, 128)), _full_spec((1, 128)),
              _full_spec((128, 128))],
    out_specs=[_rows_spec(128), _rows_spec(128)],
    out_shape=[jax.ShapeDtypeStruct((N_PAD, 128), jnp.float32),
               jax.ShapeDtypeStruct((N_PAD, 128), jnp.float32)],
)

_pre = pl.pallas_call(
    _pre_body,
    grid=(_GRID,),
    in_specs=[_rows_spec(128), _rows_spec(1), _full_spec((128, 128))],
    out_specs=[_rows_spec(128), _rows_spec(1)],
    out_shape=[jax.ShapeDtypeStruct((N_PAD, 128), jnp.float32),
               jax.ShapeDtypeStruct((N_PAD, 1), jnp.float32)],
)

_post = pl.pallas_call(
    _post_body,
    grid=(_GRID,),
    in_specs=[_rows_spec(128), _rows_spec(128), _rows_spec(128), _rows_spec(1),
              _rows_spec(1), _rows_spec(1), _part_spec(128), _part_spec(128),
              _full_spec((1, 128))],
    out_specs=_rows_spec(128),
    out_shape=jax.ShapeDtypeStruct((N_PAD, 128), jnp.float32),
)

_dec = pl.pallas_call(
    _dec_body,
    grid=(_GRID,),
    in_specs=[_rows_spec(128), _full_spec((128, 40)), _full_spec((1, 40))],
    out_specs=_rows_spec(40),
    out_shape=jax.ShapeDtypeStruct((N_PAD, 40), jnp.float32),
)


def kernel(x, edge_index, W_enc, b_enc, W_conv, b_conv, W_ggs, b_ggs, W_ggq,
           b_ggq, W_skip, W_dec, b_dec):
    n = x.shape[0]
    e = edge_index.shape[1]
    pad = jnp.full((ECH_STAGE * CH - e,), n, jnp.int32)
    rowp = jnp.concatenate([edge_index[0], pad]).reshape(ECH_STAGE, CH)
    colp = jnp.concatenate([edge_index[1], pad]).reshape(ECH_STAGE, CH)
    xp = jnp.zeros((N_PAD, x.shape[1]), x.dtype).at[:n].set(x)
    zrows = jnp.zeros((N_PAD, 128), jnp.float32)

    degp = _deg_k(colp).reshape(NC, N_PAD)
    deg = (degp[0] + degp[1]).reshape(N_PAD, 1)
    X, skip = _enc(xp, W_enc, b_enc.reshape(1, -1), W_skip)
    zvec = jnp.zeros((N_PAD,), jnp.float32)
    for _ in range(2):
        Y, q = _pre(X, deg, W_conv)
        S, T, cu = _edge_k(Y, X, rowp, colp, q.reshape(N_PAD), zrows, zvec)
        cnt = cu[:N_PAD].reshape(N_PAD, 1)
        u = cu[N_PAD:].reshape(N_PAD, 1)
        X = _post(X, Y, skip, deg, cnt, u, S, T, b_conv.reshape(1, -1))
    out = _dec(X, W_dec, b_dec.reshape(1, -1))
    return out[:n]


# speedup vs baseline: 1.0880x; 1.0880x over previous
import functools

import jax
import jax.numpy as jnp
from jax import lax
from jax.experimental import pallas as pl
from jax.experimental.pallas import tpu as pltpu
from jax.experimental.pallas import tpu_sc as plsc

N_PAD = 10240
E_PAD = 327680
CH = 128
RB = 1280

_info = plsc.get_sparse_core_info()
NC, NS = _info.num_cores, _info.num_subcores
NW = NC * NS
ECH = E_PAD // CH
ECH_STAGE = ECH + 32
NCHUNK = ECH // NW
ROWS_PT = N_PAD // NS

NCH0 = 144
NCH1 = 16
PH_BUF = 24

_MESH = dict(mesh=plsc.VectorSubcoreMesh(core_axis_name="c", subcore_axis_name="s"))


@functools.partial(
    pl.kernel,
    out_type=jax.ShapeDtypeStruct((NC * N_PAD,), jnp.float32),
    scratch_types=[
        pltpu.VMEM((NCHUNK, CH), jnp.int32),
        pltpu.VMEM((CH,), jnp.float32),
        pltpu.VMEM((640,), jnp.float32),
        pltpu.VMEM_SHARED((N_PAD,), jnp.float32),
        pltpu.SemaphoreType.DMA,
    ],
    **_MESH,
)
def _deg_k(dst_hbm, out_hbm, dsts, onesb, zbuf, acc, sem):
    cid = lax.axis_index("c")
    sid = lax.axis_index("s")
    wid = sid * NC + cid
    for j in range(CH // 16):
        onesb[pl.ds(j * 16, 16)] = jnp.ones((16,), jnp.float32)
    for j in range(640 // 16):
        zbuf[pl.ds(j * 16, 16)] = jnp.zeros((16,), jnp.float32)
    base_r = sid * ROWS_PT
    pltpu.sync_copy(dst_hbm.at[pl.ds(wid * NCHUNK, NCHUNK)], dsts)
    pltpu.sync_copy(zbuf.at[pl.ds(0, ROWS_PT)], acc.at[pl.ds(base_r, ROWS_PT)])
    plsc.subcore_barrier()

    def body(g, carry):
        pltpu.async_copy(onesb, acc.at[dsts.at[g]], sem, add=True)
        return carry

    lax.fori_loop(0, NCHUNK, body, 0)

    def drain(g, carry):
        pltpu.make_async_copy(onesb, acc.at[dsts.at[g]], sem).wait()
        return carry

    lax.fori_loop(0, NCHUNK, drain, 0)
    plsc.subcore_barrier()
    obase = pl.multiple_of(cid * N_PAD + base_r, 8)
    pltpu.sync_copy(acc.at[pl.ds(base_r, ROWS_PT)],
                    out_hbm.at[pl.ds(obase, ROWS_PT)])


CUPH0 = 6
CUPH1 = 4
CUCH = 16


@functools.partial(
    pl.kernel,
    out_type=[jax.ShapeDtypeStruct((NC, N_PAD, 128), jnp.float32),
              jax.ShapeDtypeStruct((NC, N_PAD, 128), jnp.float32),
              jax.ShapeDtypeStruct((NC * 2 * N_PAD,), jnp.float32)],
    scratch_types=[
        pltpu.VMEM((PH_BUF, CH), jnp.int32),
        pltpu.VMEM((PH_BUF, CH), jnp.int32),
        pltpu.VMEM((CH, 128), jnp.float32),
        pltpu.VMEM((CH, 128), jnp.float32),
        pltpu.VMEM((CH,), jnp.float32),
        pltpu.VMEM((CH,), jnp.float32),
        pltpu.VMEM((CH,), jnp.float32),
        pltpu.VMEM_SHARED((N_PAD, 128), jnp.float32),
        pltpu.VMEM_SHARED((N_PAD,), jnp.float32),
        pltpu.VMEM_SHARED((N_PAD,), jnp.float32),
        pltpu.SemaphoreType.DMA,
        pltpu.SemaphoreType.DMA,
        pltpu.SemaphoreType.DMA,
        pltpu.SemaphoreType.DMA,
        pltpu.SemaphoreType.DMA,
        pltpu.SemaphoreType.DMA,
        pltpu.SemaphoreType.DMA,
        pltpu.SemaphoreType.DMA,
        pltpu.SemaphoreType.DMA,
        pltpu.SemaphoreType.DMA,
    ],
    **_MESH,
)
def _edge_k(Y_hbm, X_hbm, row_hbm, col_hbm, q_hbm, zero_hbm, zvec_hbm,
            S_out, T_out, cu_out, srcs, dsts, rows0, rows1, onesb, ust0, ust1,
            acc, acc_c, acc_u, gsem0, gsem1, ssem0, ssem1, zsem, csem,
            qsem0, qsem1, usem0, usem1):
    cid = lax.axis_index("c")
    sid = lax.axis_index("s")
    is0 = cid == 0
    base_r = sid * ROWS_PT

    def zero_acc():
        pltpu.async_copy(zero_hbm.at[pl.ds(base_r, ROWS_PT)],
                         acc.at[pl.ds(base_r, ROWS_PT)], zsem)
        pltpu.make_async_copy(zero_hbm.at[pl.ds(base_r, ROWS_PT)],
                              acc.at[pl.ds(base_r, ROWS_PT)], zsem).wait()

    zero_acc()
    for j in range(CH // 16):
        onesb[pl.ds(j * 16, 16)] = jnp.ones((16,), jnp.float32)
    pltpu.sync_copy(zvec_hbm.at[pl.ds(base_r, ROWS_PT)],
                    acc_c.at[pl.ds(base_r, ROWS_PT)])
    pltpu.sync_copy(zvec_hbm.at[pl.ds(base_r, ROWS_PT)],
                    acc_u.at[pl.ds(base_r, ROWS_PT)])
    plsc.subcore_barrier()

    def run_phase(table_hbm, src_hbm, dst_hbm, pstart, ph):
        pltpu.sync_copy(src_hbm.at[pl.ds(pstart, PH_BUF)], srcs)
        pltpu.sync_copy(dst_hbm.at[pl.ds(pstart, PH_BUF)], dsts)
        pltpu.async_copy(table_hbm.at[srcs.at[0]], rows0, gsem0)

        def body(j, carry):
            g = 2 * j
            pltpu.make_async_copy(table_hbm.at[srcs.at[g]], rows0, gsem0).wait()
            pltpu.async_copy(rows0, acc.at[dsts.at[g]], ssem0, add=True)

            @pl.when(j > 0)
            def _():
                pltpu.make_async_copy(rows1, acc.at[dsts.at[g - 1]], ssem1).wait()

            pltpu.async_copy(table_hbm.at[srcs.at[g + 1]], rows1, gsem1)
            pltpu.make_async_copy(table_hbm.at[srcs.at[g + 1]], rows1, gsem1).wait()
            pltpu.async_copy(rows1, acc.at[dsts.at[g + 1]], ssem1, add=True)
            pltpu.make_async_copy(rows0, acc.at[dsts.at[g]], ssem0).wait()

            @pl.when(2 * j + 2 < ph)
            def _():
                pltpu.async_copy(table_hbm.at[srcs.at[g + 2]], rows0, gsem0)

            return carry

        lax.fori_loop(0, ph // 2, body, 0)
        pltpu.make_async_copy(rows1, acc.at[dsts.at[ph - 1]], ssem1).wait()

    def run_pass(table_hbm, src_hbm, dst_hbm, out_hbm):
        @pl.when(is0)
        def _():
            for p in range(NCH0 // PH_BUF):
                run_phase(table_hbm, src_hbm, dst_hbm,
                          pl.multiple_of(sid * NCH0 + p * PH_BUF, 8), PH_BUF)

        @pl.when(jnp.logical_not(is0))
        def _():
            run_phase(table_hbm, src_hbm, dst_hbm,
                      pl.multiple_of(NS * NCH0 + sid * NCH1, 8), NCH1)

        plsc.subcore_barrier()
        pltpu.sync_copy(acc.at[pl.ds(base_r, ROWS_PT)],
                        out_hbm.at[cid, pl.ds(base_r, ROWS_PT)])

    run_pass(Y_hbm, row_hbm, col_hbm, S_out)
    zero_acc()
    plsc.subcore_barrier()
    run_pass(X_hbm, col_hbm, row_hbm, T_out)

    def run_cu(nph, cslab):
        for cp in range(nph):
            cstart = cslab + cp * CUCH
            pltpu.sync_copy(row_hbm.at[pl.ds(cstart, CUCH)], srcs.at[pl.ds(0, CUCH)])
            pltpu.sync_copy(col_hbm.at[pl.ds(cstart, CUCH)], dsts.at[pl.ds(0, CUCH)])
            pltpu.async_copy(q_hbm.at[dsts.at[0]], ust0, qsem0)

            def ubody(j, carry):
                g = 2 * j
                pltpu.async_copy(onesb, acc_c.at[srcs.at[g]], csem, add=True)
                pltpu.async_copy(onesb, acc_c.at[srcs.at[g + 1]], csem, add=True)
                pltpu.make_async_copy(q_hbm.at[dsts.at[g]], ust0, qsem0).wait()
                pltpu.async_copy(ust0, acc_u.at[srcs.at[g]], usem0, add=True)

                @pl.when(j > 0)
                def _():
                    pltpu.make_async_copy(ust1, acc_u.at[srcs.at[g - 1]], usem1).wait()

                pltpu.async_copy(q_hbm.at[dsts.at[g + 1]], ust1, qsem1)
                pltpu.make_async_copy(q_hbm.at[dsts.at[g + 1]], ust1, qsem1).wait()
                pltpu.async_copy(ust1, acc_u.at[srcs.at[g + 1]], usem1, add=True)
                pltpu.make_async_copy(ust0, acc_u.at[srcs.at[g]], usem0).wait()

                @pl.when(2 * j + 2 < CUCH)
                def _():
                    pltpu.async_copy(q_hbm.at[dsts.at[g + 2]], ust0, qsem0)

                return carry

            lax.fori_loop(0, CUCH // 2, ubody, 0)
            pltpu.make_async_copy(ust1, acc_u.at[srcs.at[CUCH - 1]], usem1).wait()

            def cdrain(g, carry):
                pltpu.make_async_copy(onesb, acc_c.at[srcs.at[g]], csem).wait()
                return carry

            lax.fori_loop(0, CUCH, cdrain, 0)

    @pl.when(is0)
    def _():
        run_cu(CUPH0, sid * (CUPH0 * CUCH))

    @pl.when(jnp.logical_not(is0))
    def _():
        run_cu(CUPH1, NS * CUPH0 * CUCH + sid * (CUPH1 * CUCH))

    plsc.subcore_barrier()
    obase = pl.multiple_of(cid * 2 * N_PAD + base_r, 8)
    pltpu.sync_copy(acc_c.at[pl.ds(base_r, ROWS_PT)],
                    cu_out.at[pl.ds(obase, ROWS_PT)])
    obase_u = pl.multiple_of(cid * 2 * N_PAD + N_PAD + base_r, 8)
    pltpu.sync_copy(acc_u.at[pl.ds(base_r, ROWS_PT)],
                    cu_out.at[pl.ds(obase_u, ROWS_PT)])


def _enc_body(x_ref, we_ref, be_ref, ws_ref, X_ref, skip_ref):
    X = jnp.maximum(
        jnp.dot(x_ref[...], we_ref[...], preferred_element_type=jnp.float32)
        + be_ref[...], 0.0)
    X_ref[...] = X
    skip_ref[...] = jnp.dot(X, ws_ref[...], preferred_element_type=jnp.float32)


def _pre_body(X_ref, deg_ref, wc_ref, Y_ref, q_ref):
    dinv = lax.rsqrt(deg_ref[...] + 1.0)
    X = X_ref[...]
    XW = jnp.dot(X, wc_ref[...], preferred_element_type=jnp.float32)
    Y_ref[...] = dinv * XW
    q_ref[...] = jnp.sum(X * X, axis=1, keepdims=True)


def _post_body(X_ref, Y_ref, skip_ref, deg_ref, cnt_ref, u_ref, S_ref, T_ref,
               bc_ref, Xn_ref):
    dinv = lax.rsqrt(deg_ref[...] + 1.0)
    X = X_ref[...]
    S = S_ref[0] + S_ref[1]
    Xagg = jnp.maximum(dinv * (S + Y_ref[...]) + bc_ref[...], 0.0)
    T = T_ref[0] + T_ref[1]
    cnt = cnt_ref[...]
    q = jnp.sum(X * X, axis=1, keepdims=True)
    sd = cnt * q + u_ref[...] - 2.0 * jnp.sum(X * T, axis=1, keepdims=True)
    g = jnp.tanh(sd / jnp.maximum(cnt, 1.0))
    Xn_ref[...] = (X + g * (Xagg + skip_ref[...])) / (1.0 + 2.0 * g)


def _dec_body(X_ref, wd_ref, bd_ref, out_ref):
    out_ref[...] = (
        jnp.dot(X_ref[...], wd_ref[...], preferred_element_type=jnp.float32)
        + bd_ref[...])


def _rows_spec(w):
    return pl.BlockSpec((RB, w), lambda i: (i, 0))


def _full_spec(shape):
    return pl.BlockSpec(shape, lambda i: tuple(0 for _ in shape))


def _part_spec(w):
    return pl.BlockSpec((NC, RB, w), lambda i: (0, i, 0))


_GRID = N_PAD // RB

_enc = pl.pallas_call(
    _enc_body,
    grid=(_GRID,),
    in_specs=[_rows_spec(128), _full_spec((128, 128)), _full_spec((1, 128)),
              _full_spec((128, 128))],
    out_specs=[_rows_spec(128), _rows_spec(128)],
    out_shape=[jax.ShapeDtypeStruct((N_PAD, 128), jnp.float32),
               jax.ShapeDtypeStruct((N_PAD, 128), jnp.float32)],
)

_pre = pl.pallas_call(
    _pre_body,
    grid=(_GRID,),
    in_specs=[_rows_spec(128), _rows_spec(1), _full_spec((128, 128))],
    out_specs=[_rows_spec(128), _rows_spec(1)],
    out_shape=[jax.ShapeDtypeStruct((N_PAD, 128), jnp.float32),
               jax.ShapeDtypeStruct((N_PAD, 1), jnp.float32)],
)

_post = pl.pallas_call(
    _post_body,
    grid=(_GRID,),
    in_specs=[_rows_spec(128), _rows_spec(128), _rows_spec(128), _rows_spec(1),
              _rows_spec(1), _rows_spec(1), _part_spec(128), _part_spec(128),
              _full_spec((1, 128))],
    out_specs=_rows_spec(128),
    out_shape=jax.ShapeDtypeStruct((N_PAD, 128), jnp.float32),
)

_dec = pl.pallas_call(
    _dec_body,
    grid=(_GRID,),
    in_specs=[_rows_spec(128), _full_spec((128, 40)), _full_spec((1, 40))],
    out_specs=_rows_spec(40),
    out_shape=jax.ShapeDtypeStruct((N_PAD, 40), jnp.float32),
)


def kernel(x, edge_index, W_enc, b_enc, W_conv, b_conv, W_ggs, b_ggs, W_ggq,
           b_ggq, W_skip, W_dec, b_dec):
    n = x.shape[0]
    e = edge_index.shape[1]
    pad = jnp.full((ECH_STAGE * CH - e,), n, jnp.int32)
    rowp = jnp.concatenate([edge_index[0], pad]).reshape(ECH_STAGE, CH)
    colp = jnp.concatenate([edge_index[1], pad]).reshape(ECH_STAGE, CH)
    xp = jnp.zeros((N_PAD, x.shape[1]), x.dtype).at[:n].set(x)
    zrows = jnp.zeros((N_PAD, 128), jnp.float32)

    degp = _deg_k(colp).reshape(NC, N_PAD)
    deg = (degp[0] + degp[1]).reshape(N_PAD, 1)
    X, skip = _enc(xp, W_enc, b_enc.reshape(1, -1), W_skip)
    zvec = jnp.zeros((N_PAD,), jnp.float32)
    for _ in range(2):
        Y, q = _pre(X, deg, W_conv)
        S, T, cu = _edge_k(Y, X, rowp, colp, q.reshape(N_PAD), zrows, zvec)
        cus = cu.reshape(NC, 2, N_PAD)
        cusum = cus[0] + cus[1]
        cnt = cusum[0].reshape(N_PAD, 1)
        u = cusum[1].reshape(N_PAD, 1)
        X = _post(X, Y, skip, deg, cnt, u, S, T, b_conv.reshape(1, -1))
    out = _dec(X, W_dec, b_dec.reshape(1, -1))
    return out[:n]
